# baseline (speedup 1.0000x reference)
import functools

import jax
import jax.numpy as jnp
from jax import lax
from jax.experimental import pallas as pl
from jax.experimental.pallas import tpu as pltpu
from jax.experimental.pallas import tpu_sc as plsc

_NBUF = 2


def _eye128():
    r = lax.broadcasted_iota(jnp.int32, (128, 128), 0)
    c = lax.broadcasted_iota(jnp.int32, (128, 128), 1)
    return jnp.where(r == c, 1.0, 0.0).astype(jnp.float32)


def _mxu_t(x, ident):
    return lax.dot_general(
        ident, x, (((1,), (1,)), ((), ())), preferred_element_type=jnp.float32
    )


def _pack_table(wT, V, D):
    CI = 32768
    NB = (V + CI - 1) // CI
    NQ = CI // 512

    def body(x_ref, o_ref):
        x = x_ref[...]
        ident = _eye128()
        zs = []
        for q in range(NQ):
            v = jnp.concatenate(
                [
                    x[:, 128 * (4 * q + k) : 128 * (4 * q + k) + 128]
                    for k in range(4)
                ],
                axis=0,
            )
            zs.append(_mxu_t(v, ident) if q % 2 == 0 else v.T)
        o_ref[...] = jnp.concatenate(zs, axis=0)

    return pl.pallas_call(
        body,
        grid=(NB,),
        in_specs=[pl.BlockSpec((D, CI), lambda i: (0, i))],
        out_specs=pl.BlockSpec((CI // 4, 4 * D), lambda i: (i, 0)),
        out_shape=jax.ShapeDtypeStruct((NB * CI // 4, 4 * D), jnp.float32),
    )(wT)


def _gather_fn(B, D, VR, CH, num_ch, b_per_w, num_cores):
    mesh = plsc.VectorSubcoreMesh(core_axis_name="c", subcore_axis_name="s")

    @functools.partial(
        pl.kernel,
        mesh=mesh,
        out_type=jax.ShapeDtypeStruct((B, D), jnp.float32),
        scratch_types=[
            pltpu.VMEM((b_per_w,), jnp.int32),
            [pltpu.VMEM((CH, D), jnp.float32) for _ in range(_NBUF)],
            [pltpu.VMEM((CH,), jnp.int32) for _ in range(_NBUF)],
            [pltpu.SemaphoreType.DMA for _ in range(_NBUF)],
            [pltpu.SemaphoreType.DMA for _ in range(_NBUF)],
        ],
        compiler_params=pltpu.CompilerParams(use_tc_tiling_on_sc=False),
    )
    def k(idx_hbm, table_hbm, out_hbm, idx_v, rows, sidxs, g_sems, s_sems):
        wid = lax.axis_index("s") * num_cores + lax.axis_index("c")
        base = wid * b_per_w
        b_base = wid * (b_per_w // 28)
        pltpu.sync_copy(idx_hbm.at[pl.ds(base, b_per_w)], idx_v)

        gathers = [None] * num_ch
        stores = [None] * num_ch

        def fire_gather(i):
            s = i % _NBUF
            gathers[i] = pltpu.async_copy(
                table_hbm.at[idx_v.at[pl.ds(i * CH, CH)]], rows[s], g_sems[s]
            )
            b0 = b_base + i * (CH // 28)

            def sv(v, _):
                row = lax.iota(jnp.int32, 16) + v * 16
                bp = (row * 2341) >> 16
                f = row - bp * 28
                sidx = ((f >> 2) << 16) + ((b0 + bp) << 2) + (f & 3)
                sidxs[s][pl.ds(v * 16, 16)] = sidx
                return _

            lax.fori_loop(0, CH // 16, sv, 0)

        for b in range(min(_NBUF, num_ch)):
            fire_gather(b)
        for i in range(num_ch):
            s = i % _NBUF
            gathers[i].wait()
            stores[i] = pltpu.async_copy(
                rows[s], out_hbm.at[sidxs[s]], s_sems[s]
            )
            nxt = i + _NBUF
            if nxt < num_ch:
                stores[i].wait()
                fire_gather(nxt)
        for i in range(max(0, num_ch - _NBUF), num_ch):
            stores[i].wait()

    return k


def _to_native_out(x7, Bt, F, D):
    BB = 2048
    FD = F * D
    NBLK = Bt // BB

    def body(x_ref, o_ref):
        x = x_ref[...]
        o_ref[...] = x.T

    return pl.pallas_call(
        body,
        grid=(7, NBLK),
        in_specs=[pl.BlockSpec((BB, 128), lambda s, i: (s * NBLK + i, 0))],
        out_specs=pl.BlockSpec((128, BB), lambda s, i: (s, i)),
        out_shape=jax.ShapeDtypeStruct((FD, Bt), jnp.float32),
    )(x7)


def kernel(indices, weights):
    Bt, F = indices.shape
    V, D = weights.shape
    FP = 28
    B2 = Bt * FP

    t4 = _pack_table(weights.T, V, D)
    VR = t4.shape[0] * 4
    table = t4.reshape(VR, D)

    idx = indices.astype(jnp.int32)
    idx = jnp.pad(idx, ((0, 0), (0, FP - F)), mode="wrap")
    r = idx - (idx % 512) + 4 * (idx % 128) + (idx // 128) % 4
    idx_flat = r.reshape(B2)

    info = plsc.get_sparse_core_info()
    NW = info.num_cores * info.num_subcores
    b_per_w = B2 // NW
    CH = 896
    num_ch = b_per_w // CH
    assert b_per_w % CH == 0 and B2 % NW == 0

    flat = _gather_fn(B2, D, VR, CH, num_ch, b_per_w, info.num_cores)(
        idx_flat, table
    )

    phys = _to_native_out(flat.reshape(7 * Bt, 128), Bt, F, D)
    return jnp.transpose(phys.reshape(F, D, Bt), (2, 0, 1))


# speedup vs baseline: 1.0834x; 1.0834x over previous
import functools

import jax
import jax.numpy as jnp
from jax import lax
from jax.experimental import pallas as pl
from jax.experimental.pallas import tpu as pltpu
from jax.experimental.pallas import tpu_sc as plsc

_NBUF = 2


def _eye128():
    r = lax.broadcasted_iota(jnp.int32, (128, 128), 0)
    c = lax.broadcasted_iota(jnp.int32, (128, 128), 1)
    return jnp.where(r == c, 1.0, 0.0).astype(jnp.float32)


def _mxu_t(x, ident):
    return lax.dot_general(
        ident, x, (((1,), (1,)), ((), ())), preferred_element_type=jnp.float32
    )


def _pack_table(wT, V, D):
    CI = 32768
    NB = (V + CI - 1) // CI
    NQ = CI // 512

    def body(x_ref, o_ref):
        x = x_ref[...]
        ident = _eye128()
        zs = []
        for q in range(NQ):
            v = jnp.concatenate(
                [
                    x[:, 128 * (4 * q + k) : 128 * (4 * q + k) + 128]
                    for k in range(4)
                ],
                axis=0,
            )
            zs.append(_mxu_t(v, ident) if q % 2 == 0 else v.T)
        o_ref[...] = jnp.concatenate(zs, axis=0)

    return pl.pallas_call(
        body,
        grid=(NB,),
        in_specs=[pl.BlockSpec((D, CI), lambda i: (0, i))],
        out_specs=pl.BlockSpec((CI // 4, 4 * D), lambda i: (i, 0)),
        out_shape=jax.ShapeDtypeStruct((NB * CI // 4, 4 * D), jnp.float32),
    )(wT)


def _gather_fn(B, D, VR, CH, num_ch, b_per_w, num_cores):
    mesh = plsc.VectorSubcoreMesh(core_axis_name="c", subcore_axis_name="s")

    @functools.partial(
        pl.kernel,
        mesh=mesh,
        out_type=jax.ShapeDtypeStruct((B, D), jnp.float32),
        scratch_types=[
            pltpu.VMEM((b_per_w,), jnp.int32),
            [pltpu.VMEM((CH, D), jnp.float32) for _ in range(_NBUF)],
            [pltpu.VMEM((CH,), jnp.int32) for _ in range(_NBUF)],
            [pltpu.SemaphoreType.DMA for _ in range(_NBUF)],
            [pltpu.SemaphoreType.DMA for _ in range(_NBUF)],
        ],
        compiler_params=pltpu.CompilerParams(use_tc_tiling_on_sc=False),
    )
    def k(idx_hbm, table_hbm, out_hbm, idx_v, rows, sidxs, g_sems, s_sems):
        wid = lax.axis_index("s") * num_cores + lax.axis_index("c")
        base = wid * b_per_w
        b_base = wid * (b_per_w // 28)
        pltpu.sync_copy(idx_hbm.at[pl.ds(base, b_per_w)], idx_v)

        gathers = [None] * num_ch
        stores = [None] * num_ch

        def fire_gather(i):
            s = i % _NBUF
            gathers[i] = pltpu.async_copy(
                table_hbm.at[idx_v.at[pl.ds(i * CH, CH)]], rows[s], g_sems[s]
            )
            b0 = b_base + i * (CH // 28)

            def sv(v, _):
                row = lax.iota(jnp.int32, 16) + v * 16
                bp = (row * 2341) >> 16
                f = row - bp * 28
                sidx = ((f >> 2) << 16) + ((b0 + bp) << 2) + (f & 3)
                sidxs[s][pl.ds(v * 16, 16)] = sidx
                return _

            lax.fori_loop(0, CH // 16, sv, 0)

        for b in range(min(_NBUF, num_ch)):
            fire_gather(b)
        for i in range(num_ch):
            s = i % _NBUF
            gathers[i].wait()
            stores[i] = pltpu.async_copy(
                rows[s], out_hbm.at[sidxs[s]], s_sems[s]
            )
            nxt = i + _NBUF
            if nxt < num_ch:
                stores[i].wait()
                fire_gather(nxt)
        for i in range(max(0, num_ch - _NBUF), num_ch):
            stores[i].wait()

    return k


def _to_native_out(x7, Bt, F, D):
    BB = 4096
    FD = F * D
    NBLK = Bt // BB

    def body(x_ref, o_ref):
        x = x_ref[...]
        o_ref[...] = x.T

    return pl.pallas_call(
        body,
        grid=(7, NBLK),
        in_specs=[pl.BlockSpec((BB, 128), lambda s, i: (s * NBLK + i, 0))],
        out_specs=pl.BlockSpec((128, BB), lambda s, i: (s, i)),
        out_shape=jax.ShapeDtypeStruct((FD, Bt), jnp.float32),
    )(x7)


def kernel(indices, weights):
    Bt, F = indices.shape
    V, D = weights.shape
    FP = 28
    B2 = Bt * FP

    t4 = _pack_table(weights.T, V, D)
    VR = t4.shape[0] * 4
    table = t4.reshape(VR, D)

    idx = indices.astype(jnp.int32)
    idx = jnp.pad(idx, ((0, 0), (0, FP - F)), mode="wrap")
    r = idx - (idx % 512) + 4 * (idx % 128) + (idx // 128) % 4
    idx_flat = r.reshape(B2)

    info = plsc.get_sparse_core_info()
    NW = info.num_cores * info.num_subcores
    b_per_w = B2 // NW
    CH = 896
    num_ch = b_per_w // CH
    assert b_per_w % CH == 0 and B2 % NW == 0

    flat = _gather_fn(B2, D, VR, CH, num_ch, b_per_w, info.num_cores)(
        idx_flat, table
    )

    phys = _to_native_out(flat.reshape(7 * Bt, 128), Bt, F, D)
    return jnp.transpose(phys.reshape(F, D, Bt), (2, 0, 1))


# speedup vs baseline: 1.1258x; 1.0392x over previous
import functools

import jax
import jax.numpy as jnp
from jax import lax
from jax.experimental import pallas as pl
from jax.experimental.pallas import tpu as pltpu
from jax.experimental.pallas import tpu_sc as plsc

_NBUF = 2


def _eye128():
    r = lax.broadcasted_iota(jnp.int32, (128, 128), 0)
    c = lax.broadcasted_iota(jnp.int32, (128, 128), 1)
    return jnp.where(r == c, 1.0, 0.0).astype(jnp.float32)


def _mxu_t(x, ident):
    return lax.dot_general(
        ident, x, (((1,), (1,)), ((), ())), preferred_element_type=jnp.float32
    )


def _pack_table(wT, V, D):
    CI = 65536
    NB = (V + CI - 1) // CI
    NQ = CI // 512

    def body(x_ref, o_ref):
        x = x_ref[...]
        ident = _eye128()
        zs = []
        for q in range(NQ):
            v = jnp.concatenate(
                [
                    x[:, 128 * (4 * q + k) : 128 * (4 * q + k) + 128]
                    for k in range(4)
                ],
                axis=0,
            )
            zs.append(_mxu_t(v, ident) if q % 2 == 0 else v.T)
        o_ref[...] = jnp.concatenate(zs, axis=0)

    return pl.pallas_call(
        body,
        grid=(NB,),
        in_specs=[pl.BlockSpec((D, CI), lambda i: (0, i))],
        out_specs=pl.BlockSpec((CI // 4, 4 * D), lambda i: (i, 0)),
        out_shape=jax.ShapeDtypeStruct((NB * CI // 4, 4 * D), jnp.float32),
    )(wT)


def _gather_fn(B, D, VR, CH, num_ch, b_per_w, num_cores):
    mesh = plsc.VectorSubcoreMesh(core_axis_name="c", subcore_axis_name="s")

    @functools.partial(
        pl.kernel,
        mesh=mesh,
        out_type=jax.ShapeDtypeStruct((B, D), jnp.float32),
        scratch_types=[
            pltpu.VMEM((b_per_w,), jnp.int32),
            [pltpu.VMEM((CH, D), jnp.float32) for _ in range(_NBUF)],
            [pltpu.VMEM((CH,), jnp.int32) for _ in range(_NBUF)],
            [pltpu.SemaphoreType.DMA for _ in range(_NBUF)],
            [pltpu.SemaphoreType.DMA for _ in range(_NBUF)],
        ],
        compiler_params=pltpu.CompilerParams(use_tc_tiling_on_sc=False),
    )
    def k(idx_hbm, table_hbm, out_hbm, idx_v, rows, sidxs, g_sems, s_sems):
        wid = lax.axis_index("s") * num_cores + lax.axis_index("c")
        base = wid * b_per_w
        b_base = wid * (b_per_w // 28)
        pltpu.sync_copy(idx_hbm.at[pl.ds(base, b_per_w)], idx_v)

        gathers = [None] * num_ch
        stores = [None] * num_ch

        def fire_gather(i):
            s = i % _NBUF
            gathers[i] = pltpu.async_copy(
                table_hbm.at[idx_v.at[pl.ds(i * CH, CH)]], rows[s], g_sems[s]
            )
            b0 = b_base + i * (CH // 28)

            def sv(v, _):
                row = lax.iota(jnp.int32, 16) + v * 16
                bp = (row * 2341) >> 16
                f = row - bp * 28
                sidx = ((f >> 2) << 16) + ((b0 + bp) << 2) + (f & 3)
                sidxs[s][pl.ds(v * 16, 16)] = sidx
                return _

            lax.fori_loop(0, CH // 16, sv, 0)

        for b in range(min(_NBUF, num_ch)):
            fire_gather(b)
        for i in range(num_ch):
            s = i % _NBUF
            gathers[i].wait()
            stores[i] = pltpu.async_copy(
                rows[s], out_hbm.at[sidxs[s]], s_sems[s]
            )
            nxt = i + _NBUF
            if nxt < num_ch:
                stores[i].wait()
                fire_gather(nxt)
        for i in range(max(0, num_ch - _NBUF), num_ch):
            stores[i].wait()

    return k


def _to_native_out(x7, Bt, F, D):
    BB = 8192
    FD = F * D
    NBLK = Bt // BB

    def body(x_ref, o_ref):
        x = x_ref[...]
        o_ref[...] = x.T

    return pl.pallas_call(
        body,
        grid=(7, NBLK),
        in_specs=[pl.BlockSpec((BB, 128), lambda s, i: (s * NBLK + i, 0))],
        out_specs=pl.BlockSpec((128, BB), lambda s, i: (s, i)),
        out_shape=jax.ShapeDtypeStruct((FD, Bt), jnp.float32),
    )(x7)


def kernel(indices, weights):
    Bt, F = indices.shape
    V, D = weights.shape
    FP = 28
    B2 = Bt * FP

    t4 = _pack_table(weights.T, V, D)
    VR = t4.shape[0] * 4
    table = t4.reshape(VR, D)

    idx = indices.astype(jnp.int32)
    idx = jnp.pad(idx, ((0, 0), (0, FP - F)), mode="wrap")
    r = idx - (idx % 512) + 4 * (idx % 128) + (idx // 128) % 4
    idx_flat = r.reshape(B2)

    info = plsc.get_sparse_core_info()
    NW = info.num_cores * info.num_subcores
    b_per_w = B2 // NW
    CH = 896
    num_ch = b_per_w // CH
    assert b_per_w % CH == 0 and B2 % NW == 0

    flat = _gather_fn(B2, D, VR, CH, num_ch, b_per_w, info.num_cores)(
        idx_flat, table
    )

    phys = _to_native_out(flat.reshape(7 * Bt, 128), Bt, F, D)
    return jnp.transpose(phys.reshape(F, D, Bt), (2, 0, 1))


# speedup vs baseline: 1.2155x; 1.0797x over previous
import functools

import jax
import jax.numpy as jnp
from jax import lax
from jax.experimental import pallas as pl
from jax.experimental.pallas import tpu as pltpu
from jax.experimental.pallas import tpu_sc as plsc

_NBUF = 2


def _eye128():
    r = lax.broadcasted_iota(jnp.int32, (128, 128), 0)
    c = lax.broadcasted_iota(jnp.int32, (128, 128), 1)
    return jnp.where(r == c, 1.0, 0.0).astype(jnp.float32)


def _mxu_t(x, ident):
    return lax.dot_general(
        ident, x, (((1,), (1,)), ((), ())), preferred_element_type=jnp.float32
    )


def _pack_table(wT, V, D):
    CI = 65536
    NB = (V + CI - 1) // CI
    NQ = CI // 512

    def body(x_ref, o_ref):
        x = x_ref[...]
        ident = _eye128()
        zs = []
        for q in range(NQ):
            v = jnp.concatenate(
                [
                    x[:, 128 * (4 * q + k) : 128 * (4 * q + k) + 128]
                    for k in range(4)
                ],
                axis=0,
            )
            zs.append(_mxu_t(v, ident) if q % 2 == 0 else v.T)
        o_ref[...] = jnp.concatenate(zs, axis=0)

    return pl.pallas_call(
        body,
        grid=(NB,),
        in_specs=[pl.BlockSpec((D, CI), lambda i: (0, i))],
        out_specs=pl.BlockSpec((CI // 4, 4 * D), lambda i: (i, 0)),
        out_shape=jax.ShapeDtypeStruct((NB * CI // 4, 4 * D), jnp.float32),
    )(wT)


def _prep_idx(idxT, Bt, F, FP):
    def body(x_ref, o_ref):
        x = x_ref[...]
        xp = jnp.concatenate([x, x[0 : FP - F, :]], axis=0)
        t = xp - (xp % 512) + 4 * (xp % 128) + (xp // 128) % 4
        o_ref[...] = t.reshape(FP, Bt // 128, 128)

    return pl.pallas_call(
        body,
        grid=(1,),
        in_specs=[pl.BlockSpec((F, Bt), lambda i: (0, 0))],
        out_specs=pl.BlockSpec((FP, Bt // 128, 128), lambda i: (0, 0, 0)),
        out_shape=jax.ShapeDtypeStruct((FP, Bt // 128, 128), jnp.int32),
    )(idxT)


def _gather_fn(B, D, VR, CH, num_ch, b_per_w, num_cores):
    mesh = plsc.VectorSubcoreMesh(core_axis_name="c", subcore_axis_name="s")

    @functools.partial(
        pl.kernel,
        mesh=mesh,
        out_type=jax.ShapeDtypeStruct((B, D), jnp.float32),
        scratch_types=[
            pltpu.VMEM((b_per_w,), jnp.int32),
            [pltpu.VMEM((CH, D), jnp.float32) for _ in range(_NBUF)],
            [pltpu.VMEM((CH,), jnp.int32) for _ in range(_NBUF)],
            [pltpu.SemaphoreType.DMA for _ in range(_NBUF)],
            [pltpu.SemaphoreType.DMA for _ in range(_NBUF)],
        ],
        compiler_params=pltpu.CompilerParams(use_tc_tiling_on_sc=False),
    )
    def k(idx_hbm, table_hbm, out_hbm, idx_v, rows, sidxs, g_sems, s_sems):
        wid = lax.axis_index("s") * num_cores + lax.axis_index("c")
        base = wid * b_per_w
        pltpu.sync_copy(idx_hbm.at[pl.ds(base, b_per_w)], idx_v)

        gathers = [None] * num_ch
        stores = [None] * num_ch

        def fire_gather(i):
            s = i % _NBUF
            gathers[i] = pltpu.async_copy(
                table_hbm.at[idx_v.at[pl.ds(i * CH, CH)]], rows[s], g_sems[s]
            )
            p0 = base + i * CH

            def sv(v, _):
                p = lax.iota(jnp.int32, 16) + (p0 + v * 16)
                f = p >> 14
                b = p & 16383
                sidx = ((f >> 2) << 16) + (b << 2) + (f & 3)
                sidxs[s][pl.ds(v * 16, 16)] = sidx
                return _

            lax.fori_loop(0, CH // 16, sv, 0)

        for b in range(min(_NBUF, num_ch)):
            fire_gather(b)
        for i in range(num_ch):
            s = i % _NBUF
            gathers[i].wait()
            stores[i] = pltpu.async_copy(
                rows[s], out_hbm.at[sidxs[s]], s_sems[s]
            )
            nxt = i + _NBUF
            if nxt < num_ch:
                stores[i].wait()
                fire_gather(nxt)
        for i in range(max(0, num_ch - _NBUF), num_ch):
            stores[i].wait()

    return k


def _to_native_out(x7, Bt, F, D):
    BB = 8192
    FD = F * D
    NBLK = Bt // BB

    def body(x_ref, o_ref):
        x = x_ref[...]
        o_ref[...] = x.T

    return pl.pallas_call(
        body,
        grid=(7, NBLK),
        in_specs=[pl.BlockSpec((BB, 128), lambda s, i: (s * NBLK + i, 0))],
        out_specs=pl.BlockSpec((128, BB), lambda s, i: (s, i)),
        out_shape=jax.ShapeDtypeStruct((FD, Bt), jnp.float32),
    )(x7)


def kernel(indices, weights):
    Bt, F = indices.shape
    V, D = weights.shape
    FP = 28
    B2 = Bt * FP

    t4 = _pack_table(weights.T, V, D)
    VR = t4.shape[0] * 4
    table = t4.reshape(VR, D)

    idx_flat = _prep_idx(indices.T.astype(jnp.int32), Bt, F, FP).reshape(B2)

    info = plsc.get_sparse_core_info()
    NW = info.num_cores * info.num_subcores
    b_per_w = B2 // NW
    CH = 896
    num_ch = b_per_w // CH
    assert b_per_w % CH == 0 and B2 % NW == 0

    flat = _gather_fn(B2, D, VR, CH, num_ch, b_per_w, info.num_cores)(
        idx_flat, table
    )

    phys = _to_native_out(flat.reshape(7 * Bt, 128), Bt, F, D)
    return jnp.transpose(phys.reshape(F, D, Bt), (2, 0, 1))


# speedup vs baseline: 1.2317x; 1.0133x over previous
import functools

import jax
import jax.numpy as jnp
from jax import lax
from jax.experimental import pallas as pl
from jax.experimental.pallas import tpu as pltpu
from jax.experimental.pallas import tpu_sc as plsc

_NBUF = 2


def _eye128():
    r = lax.broadcasted_iota(jnp.int32, (128, 128), 0)
    c = lax.broadcasted_iota(jnp.int32, (128, 128), 1)
    return jnp.where(r == c, 1.0, 0.0).astype(jnp.float32)


def _mxu_t(x, ident):
    return lax.dot_general(
        ident, x, (((1,), (1,)), ((), ())), preferred_element_type=jnp.float32
    )


def _pack_table(wT, V, D):
    CI = 65536
    NB = (V + CI - 1) // CI
    NQ = CI // 512

    def body(x_ref, o_ref):
        x = x_ref[...]
        ident = _eye128()
        zs = []
        for q in range(NQ):
            v = jnp.concatenate(
                [
                    x[:, 128 * (4 * q + k) : 128 * (4 * q + k) + 128]
                    for k in range(4)
                ],
                axis=0,
            )
            zs.append(_mxu_t(v, ident) if q % 2 == 0 else v.T)
        o_ref[...] = jnp.concatenate(zs, axis=0)

    return pl.pallas_call(
        body,
        grid=(NB,),
        in_specs=[pl.BlockSpec((D, CI), lambda i: (0, i))],
        out_specs=pl.BlockSpec((CI // 4, 4 * D), lambda i: (i, 0)),
        out_shape=jax.ShapeDtypeStruct((NB * CI // 4, 4 * D), jnp.float32),
    )(wT)


def _prep_idx(idxT, Bt, F, FP):
    def body(x_ref, o_ref):
        x = x_ref[...]
        t = x - (x % 512) + 4 * (x % 128) + (x // 128) % 4
        o_ref[...] = t.reshape(F, Bt // 128, 128)

    return pl.pallas_call(
        body,
        grid=(1,),
        in_specs=[pl.BlockSpec((F, Bt), lambda i: (0, 0))],
        out_specs=pl.BlockSpec((F, Bt // 128, 128), lambda i: (0, 0, 0)),
        out_shape=jax.ShapeDtypeStruct((F, Bt // 128, 128), jnp.int32),
    )(idxT)


def _gather_fn_out(B, BPAD, D, VR, CH, num_ch, b_per_w, num_cores):
    mesh = plsc.VectorSubcoreMesh(core_axis_name="c", subcore_axis_name="s")

    @functools.partial(
        pl.kernel,
        mesh=mesh,
        out_type=jax.ShapeDtypeStruct((BPAD, D), jnp.float32),
        scratch_types=[
            pltpu.VMEM((b_per_w,), jnp.int32),
            [pltpu.VMEM((CH, D), jnp.float32) for _ in range(_NBUF)],
            [pltpu.VMEM((CH,), jnp.int32) for _ in range(_NBUF)],
            [pltpu.SemaphoreType.DMA for _ in range(_NBUF)],
            [pltpu.SemaphoreType.DMA for _ in range(_NBUF)],
        ],
        compiler_params=pltpu.CompilerParams(use_tc_tiling_on_sc=False),
    )
    def k(idx_hbm, table_hbm, out_hbm, idx_v, rows, sidxs, g_sems, s_sems):
        wid = lax.axis_index("s") * num_cores + lax.axis_index("c")
        base = wid * b_per_w
        pltpu.sync_copy(idx_hbm.at[pl.ds(base, b_per_w)], idx_v)

        gathers = [None] * num_ch
        stores = [None] * num_ch

        def fire_gather(i):
            s = i % _NBUF
            gathers[i] = pltpu.async_copy(
                table_hbm.at[idx_v.at[pl.ds(i * CH, CH)]], rows[s], g_sems[s]
            )
            p0 = base + i * CH

            def sv(v, _):
                p = lax.iota(jnp.int32, 16) + (p0 + v * 16)
                f = p >> 14
                b = p & 16383
                sidx = ((f >> 2) << 16) + (b << 2) + (f & 3)
                sidxs[s][pl.ds(v * 16, 16)] = sidx
                return _

            lax.fori_loop(0, CH // 16, sv, 0)

        for b in range(min(_NBUF, num_ch)):
            fire_gather(b)
        for i in range(num_ch):
            s = i % _NBUF
            gathers[i].wait()
            stores[i] = pltpu.async_copy(
                rows[s], out_hbm.at[sidxs[s]], s_sems[s]
            )
            nxt = i + _NBUF
            if nxt < num_ch:
                stores[i].wait()
                fire_gather(nxt)
        for i in range(max(0, num_ch - _NBUF), num_ch):
            stores[i].wait()

    return k


def _to_native_out(x7, Bt, F, D):
    BB = 8192
    FD = F * D
    NBLK = Bt // BB

    def body(x_ref, o_ref):
        x = x_ref[...]
        o_ref[...] = x.T

    return pl.pallas_call(
        body,
        grid=(7, NBLK),
        in_specs=[pl.BlockSpec((BB, 128), lambda s, i: (s * NBLK + i, 0))],
        out_specs=pl.BlockSpec((128, BB), lambda s, i: (s, i)),
        out_shape=jax.ShapeDtypeStruct((FD, Bt), jnp.float32),
    )(x7)


def kernel(indices, weights):
    Bt, F = indices.shape
    V, D = weights.shape
    FP = 28
    B2 = Bt * F
    BPAD = Bt * FP

    t4 = _pack_table(weights.T, V, D)
    VR = t4.shape[0] * 4
    table = t4.reshape(VR, D)

    idx_flat = _prep_idx(indices.T.astype(jnp.int32), Bt, F, FP).reshape(B2)

    info = plsc.get_sparse_core_info()
    NW = info.num_cores * info.num_subcores
    b_per_w = B2 // NW
    CH = 832
    num_ch = b_per_w // CH
    assert b_per_w % CH == 0 and B2 % NW == 0

    flat = _gather_fn_out(
        B2, BPAD, D, VR, CH, num_ch, b_per_w, info.num_cores
    )(idx_flat, table)

    phys = _to_native_out(flat.reshape(7 * Bt, 128), Bt, F, D)
    return jnp.transpose(phys.reshape(F, D, Bt), (2, 0, 1))
